# Initial kernel scaffold; baseline (speedup 1.0000x reference)
import jax
import jax.numpy as jnp
from jax.experimental import pallas as pl


def kernel(x):
    raise NotImplementedError("write your pallas kernel here")


# speedup vs baseline: 2.9782x; 2.9782x over previous
import numpy as np

import jax
import jax.numpy as jnp
from jax.experimental import pallas as pl
from jax.experimental.pallas import tpu as pltpu


def _upsample2x_kernel(x_ref, swt_ref, o_ref):
    x = x_ref[...]
    t, h, w = x.shape
    xw = jnp.dot(x.reshape(t * h, w), swt_ref[...],
                 preferred_element_type=jnp.float32)
    xw = xw.astype(o_ref.dtype).reshape(t, h, 1, 2 * w)
    o_ref[...] = jnp.broadcast_to(xw, (t, h, 2, 2 * w))


def kernel(x):
    n, c, h, w = x.shape
    out_h, out_w = 2 * h, 2 * w

    orig_dtype = x.dtype
    if not jnp.issubdtype(orig_dtype, jnp.floating):
        x = x.astype(jnp.float32)
    compute_dtype = x.dtype

    swt_np = np.zeros((w, out_w), dtype=np.float32)
    swt_np[np.arange(out_w) // 2, np.arange(out_w)] = 1.0
    swt = jnp.asarray(swt_np, dtype=compute_dtype)

    nc = n * c
    tnc = min(nc, 64)
    nc_pad = -(-nc // tnc) * tnc

    x3 = x.reshape(nc, h, w)
    if nc_pad != nc:
        x3 = jnp.pad(x3, ((0, nc_pad - nc), (0, 0), (0, 0)))

    itemsize = jnp.dtype(compute_dtype).itemsize
    cost = pl.CostEstimate(
        flops=0, transcendentals=0,
        bytes_accessed=nc * (h * w + out_h * out_w) * itemsize)

    out4 = pl.pallas_call(
        _upsample2x_kernel,
        out_shape=jax.ShapeDtypeStruct((nc_pad, h, 2, out_w), compute_dtype),
        grid=(nc_pad // tnc,),
        in_specs=[
            pl.BlockSpec((tnc, h, w), lambda i: (i, 0, 0)),
            pl.BlockSpec((w, out_w), lambda i: (0, 0)),
        ],
        out_specs=pl.BlockSpec((tnc, h, 2, out_w), lambda i: (i, 0, 0, 0)),
        compiler_params=pltpu.CompilerParams(
            dimension_semantics=("parallel",),
            vmem_limit_bytes=48 * 1024 * 1024,
        ),
        cost_estimate=cost,
    )(x3, swt)

    out = out4[:nc].reshape(n, c, out_h, out_w)
    if out.dtype != orig_dtype:
        out = out.astype(orig_dtype)
    return out


# speedup vs baseline: 3.3054x; 1.1099x over previous
import numpy as np

import jax
import jax.numpy as jnp
from jax.experimental import pallas as pl
from jax.experimental.pallas import tpu as pltpu


def _upsample2x_kernel(x_ref, swt_ref, o_ref):
    x = x_ref[...]
    t, h, w = x.shape
    xw = jnp.dot(x.reshape(t * h, w), swt_ref[...],
                 preferred_element_type=jnp.float32)
    xw = xw.astype(o_ref.dtype).reshape(t, h, 1, 2 * w)
    o_ref[...] = jnp.broadcast_to(xw, (t, h, 2, 2 * w))


def kernel(x):
    n, c, h, w = x.shape
    out_h, out_w = 2 * h, 2 * w

    orig_dtype = x.dtype
    if not jnp.issubdtype(orig_dtype, jnp.floating):
        x = x.astype(jnp.float32)
    compute_dtype = x.dtype

    swt_np = np.zeros((w, out_w), dtype=np.float32)
    swt_np[np.arange(out_w) // 2, np.arange(out_w)] = 1.0
    swt = jnp.asarray(swt_np, dtype=compute_dtype)

    nc = n * c
    tnc = min(nc, 128)
    nc_pad = -(-nc // tnc) * tnc

    x3 = x.reshape(nc, h, w)
    if nc_pad != nc:
        x3 = jnp.pad(x3, ((0, nc_pad - nc), (0, 0), (0, 0)))

    itemsize = jnp.dtype(compute_dtype).itemsize
    cost = pl.CostEstimate(
        flops=0, transcendentals=0,
        bytes_accessed=nc * (h * w + out_h * out_w) * itemsize)

    out4 = pl.pallas_call(
        _upsample2x_kernel,
        out_shape=jax.ShapeDtypeStruct((nc_pad, h, 2, out_w), compute_dtype),
        grid=(nc_pad // tnc,),
        in_specs=[
            pl.BlockSpec((tnc, h, w), lambda i: (i, 0, 0)),
            pl.BlockSpec((w, out_w), lambda i: (0, 0)),
        ],
        out_specs=pl.BlockSpec((tnc, h, 2, out_w), lambda i: (i, 0, 0, 0)),
        compiler_params=pltpu.CompilerParams(
            dimension_semantics=("parallel",),
            vmem_limit_bytes=48 * 1024 * 1024,
        ),
        cost_estimate=cost,
    )(x3, swt)

    out = out4[:nc].reshape(n, c, out_h, out_w)
    if out.dtype != orig_dtype:
        out = out.astype(orig_dtype)
    return out
